# Initial kernel scaffold; baseline (speedup 1.0000x reference)
import jax
import jax.numpy as jnp
from jax.experimental import pallas as pl


def kernel(node_feat, edge_index, W0, b0, W1, b1, W2, b2, W3, b3, W_phi, b_phi, W_rho, b_rho):
    raise NotImplementedError("write your pallas kernel here")


# speedup vs baseline: 7.0677x; 7.0677x over previous
import functools

import jax
import jax.numpy as jnp
from jax import lax
from jax.experimental import pallas as pl
from jax.experimental.pallas import tpu as pltpu
from jax.experimental.pallas import tpu_sc as plsc

_N = 10000
_E = 320000
_B = 20
_G = _N // _B
_K = 30
_OUT = 64

_NC = 2
_NS = 16
_NW = _NC * _NS
_C = 80
_CH = _E // (_NW * _C)
_NP = 10240
_RPT = _NP // _NS


def _sc_spmm(y_tab, idx4, zrow, width):
  mesh = plsc.VectorSubcoreMesh(core_axis_name="c", subcore_axis_name="s")

  @functools.partial(
      pl.kernel,
      out_type=jax.ShapeDtypeStruct((_NC, _NP, width), jnp.float32),
      mesh=mesh,
      scratch_types=[
          pltpu.VMEM_SHARED((_NP, width), jnp.float32),
          pltpu.VMEM((2, _C), jnp.int32),
          pltpu.VMEM((_C, width), jnp.float32),
          pltpu.SemaphoreType.DMA,
      ],
      compiler_params=pltpu.CompilerParams(use_tc_tiling_on_sc=False),
  )
  def k(y_hbm, idx_hbm, z_hbm, out_hbm, msg_sh, idx_v, rows_v, sem):
    c = lax.axis_index("c")
    s = lax.axis_index("s")
    wid = c * _NS + s
    pltpu.sync_copy(z_hbm.at[pl.ds(s * _RPT, _RPT)],
                    msg_sh.at[pl.ds(s * _RPT, _RPT)])
    plsc.subcore_barrier()

    def body(j, carry):
      pltpu.sync_copy(idx_hbm.at[wid, j], idx_v)
      pltpu.async_copy(y_hbm.at[idx_v.at[0]], rows_v, sem).wait()
      pltpu.sync_copy(rows_v, msg_sh.at[idx_v.at[1]], add=True)
      return carry

    lax.fori_loop(0, _CH, body, 0)
    plsc.subcore_barrier()
    pltpu.sync_copy(msg_sh.at[pl.ds(s * _RPT, _RPT)],
                    out_hbm.at[c, pl.ds(s * _RPT, _RPT)])

  return k(y_tab, idx4, zrow)


def _tc_first(x, w0):
  def k(x_ref, w_ref, o_ref):
    y = jnp.dot(x_ref[...], w_ref[...], preferred_element_type=jnp.float32)
    o_ref[...] = jnp.concatenate(
        [y,
         jnp.ones((_N, 1), jnp.float32),
         jnp.zeros((_N, 7), jnp.float32)], axis=1)

  return pl.pallas_call(
      k, out_shape=jax.ShapeDtypeStruct((_N, 40), jnp.float32))(x, w0)


def _tc_layer1(mp, y, b, w_next):
  def k(mp_ref, y_ref, b_ref, w_ref, cur_ref, yn_ref, deg_ref):
    deg = mp_ref[0, :_N, 32:33] + mp_ref[1, :_N, 32:33] + 1.0
    t = (mp_ref[0, :_N, :32] + mp_ref[1, :_N, :32]
         + y_ref[:, :32] + b_ref[...])
    cur = jnp.tanh(t / deg)
    cur_ref[...] = cur
    deg_ref[...] = deg
    yn_ref[...] = jnp.dot(cur, w_ref[...], preferred_element_type=jnp.float32)

  return pl.pallas_call(
      k,
      out_shape=(
          jax.ShapeDtypeStruct((_N, 32), jnp.float32),
          jax.ShapeDtypeStruct((_N, w_next.shape[1]), jnp.float32),
          jax.ShapeDtypeStruct((_N, 1), jnp.float32),
      ))(mp, y, b, w_next)


def _tc_layer(mp, y, b, deg, w_next):
  def k(mp_ref, y_ref, b_ref, deg_ref, w_ref, cur_ref, yn_ref):
    t = mp_ref[0, :_N] + mp_ref[1, :_N] + y_ref[...] + b_ref[...]
    cur = jnp.tanh(t / deg_ref[...])
    cur_ref[...] = cur
    yn_ref[...] = jnp.dot(cur, w_ref[...], preferred_element_type=jnp.float32)

  return pl.pallas_call(
      k,
      out_shape=(
          jax.ShapeDtypeStruct((_N, 32), jnp.float32),
          jax.ShapeDtypeStruct((_N, w_next.shape[1]), jnp.float32),
      ))(mp, y, b, deg, w_next)


def _tc_readout(cur1, cur2, cur3, mp3, y3, b3, deg, seg,
                a1, a2, a3, a4, bp, wr, br):
  def k(c1, c2, c3, mp_ref, y3_ref, b3_ref, deg_ref, seg_ref,
        a1r, a2r, a3r, a4r, bpr, wrr, brr, o_ref):
    t4 = (mp_ref[0, :_N, 0:1] + mp_ref[1, :_N, 0:1]
          + y3_ref[:, 0:1] + b3_ref[...])
    cur4 = jnp.tanh(t4 / deg_ref[...])
    t = jnp.dot(c1[...], a1r[...], preferred_element_type=jnp.float32)
    t = t + jnp.dot(c2[...], a2r[...], preferred_element_type=jnp.float32)
    t = t + jnp.dot(c3[...], a3r[...], preferred_element_type=jnp.float32)
    t = t + cur4 * a4r[...] + bpr[...]
    t = jnp.maximum(t, 0.0)
    pooled = jnp.dot(seg_ref[...], t, preferred_element_type=jnp.float32)
    o_ref[...] = jnp.dot(pooled, wrr[...],
                         preferred_element_type=jnp.float32) + brr[...]

  return pl.pallas_call(
      k,
      out_shape=jax.ShapeDtypeStruct((_B, _OUT), jnp.float32),
  )(cur1, cur2, cur3, mp3, y3, b3, deg, seg, a1, a2, a3, a4, bp, wr, br)


@jax.jit
def kernel(node_feat, edge_index, W0, b0, W1, b1, W2, b2, W3, b3,
           W_phi, b_phi, W_rho, b_rho):
  src3 = edge_index[0].reshape(_NW, _CH, _C)
  dst3 = edge_index[1].reshape(_NW, _CH, _C)
  idx4 = jnp.stack([src3, dst3], axis=2)
  z40 = jnp.zeros((_NP, 40), jnp.float32)
  z32 = jnp.zeros((_NP, 32), jnp.float32)
  z8 = jnp.zeros((_NP, 8), jnp.float32)

  y0a = _tc_first(node_feat, W0)
  mp0 = _sc_spmm(y0a, idx4, z40, 40)
  cur1, y1, deg = _tc_layer1(mp0, y0a, b0.reshape(1, 32), W1)

  mp1 = _sc_spmm(y1, idx4, z32, 32)
  cur2, y2 = _tc_layer(mp1, y1, b1.reshape(1, 32), deg, W2)
  mp2 = _sc_spmm(y2, idx4, z32, 32)
  W3p = jnp.pad(W3, ((0, 0), (0, 7)))
  cur3, y3 = _tc_layer(mp2, y2, b2.reshape(1, 32), deg, W3p)

  mp3 = _sc_spmm(y3, idx4, z8, 8)
  seg = jnp.repeat(jnp.eye(_B, dtype=jnp.float32), _G, axis=1)
  return _tc_readout(
      cur1, cur2, cur3, mp3, y3, b3.reshape(1, 1), deg, seg,
      W_phi[0:32], W_phi[32:64], W_phi[64:96], W_phi[96:97].reshape(1, _K),
      b_phi.reshape(1, _K), W_rho, b_rho.reshape(1, _OUT))


# speedup vs baseline: 15.0568x; 2.1303x over previous
import functools

import jax
import jax.numpy as jnp
from jax import lax
from jax.experimental import pallas as pl
from jax.experimental.pallas import tpu as pltpu
from jax.experimental.pallas import tpu_sc as plsc

_N = 10000
_E = 320000
_B = 20
_G = _N // _B
_K = 30
_OUT = 64

_NC = 2
_NS = 16
_NW = _NC * _NS
_C = 50
_CH = _E // (_NW * _C)
_PK = 5
_SJ = _CH // _PK
_NP = 10240
_RPT = _NP // _NS


def _sc_spmm(y_tab, idx4, zrow, width):
  mesh = plsc.VectorSubcoreMesh(core_axis_name="c", subcore_axis_name="s")

  @functools.partial(
      pl.kernel,
      out_type=jax.ShapeDtypeStruct((_NC, _NP, width), jnp.float32),
      mesh=mesh,
      scratch_types=[
          pltpu.VMEM_SHARED((_NP, width), jnp.float32),
          pltpu.VMEM((_CH, 2, _C), jnp.int32),
          pltpu.VMEM((2, _PK, _C, width), jnp.float32),
          pltpu.SemaphoreType.DMA,
          pltpu.SemaphoreType.DMA,
      ],
      compiler_params=pltpu.CompilerParams(use_tc_tiling_on_sc=False),
  )
  def k(y_hbm, idx_hbm, z_hbm, out_hbm, msg_sh, idx_v, rows_v, gsem, ssem):
    c = lax.axis_index("c")
    s = lax.axis_index("s")
    wid = c * _NS + s
    pltpu.sync_copy(idx_hbm.at[wid], idx_v)
    pltpu.sync_copy(z_hbm.at[pl.ds(s * _RPT, _RPT)],
                    msg_sh.at[pl.ds(s * _RPT, _RPT)])
    plsc.subcore_barrier()

    def body(sj, carry):
      p = lax.rem(sj, 2)

      @pl.when(sj >= 2)
      def _():
        for b in range(_PK):
          pltpu.make_async_copy(
              y_hbm.at[pl.ds(0, _C)], rows_v.at[p, b], ssem).wait()

      gds = []
      for b in range(_PK):
        gds.append(pltpu.async_copy(
            y_hbm.at[idx_v.at[sj * _PK + b, 0]], rows_v.at[p, b], gsem))
      for d in gds:
        d.wait()
      for b in range(_PK):
        pltpu.async_copy(
            rows_v.at[p, b], msg_sh.at[idx_v.at[sj * _PK + b, 1]], ssem,
            add=True)
      return carry

    lax.fori_loop(0, _SJ, body, 0)
    for b in range(2 * _PK):
      pltpu.make_async_copy(
          y_hbm.at[pl.ds(0, _C)], rows_v.at[0, b % _PK], ssem).wait()
    plsc.subcore_barrier()
    pltpu.sync_copy(msg_sh.at[pl.ds(s * _RPT, _RPT)],
                    out_hbm.at[c, pl.ds(s * _RPT, _RPT)])

  return k(y_tab, idx4, zrow)


def _tc_first(x, w0):
  def k(x_ref, w_ref, o_ref):
    y = jnp.dot(x_ref[...], w_ref[...], preferred_element_type=jnp.float32)
    o_ref[...] = jnp.concatenate(
        [y,
         jnp.ones((_N, 1), jnp.float32),
         jnp.zeros((_N, 7), jnp.float32)], axis=1)

  return pl.pallas_call(
      k, out_shape=jax.ShapeDtypeStruct((_N, 40), jnp.float32))(x, w0)


def _tc_layer1(mp, y, b, w_next):
  def k(mp_ref, y_ref, b_ref, w_ref, cur_ref, yn_ref, deg_ref):
    deg = mp_ref[0, :_N, 32:33] + mp_ref[1, :_N, 32:33] + 1.0
    t = (mp_ref[0, :_N, :32] + mp_ref[1, :_N, :32]
         + y_ref[:, :32] + b_ref[...])
    cur = jnp.tanh(t / deg)
    cur_ref[...] = cur
    deg_ref[...] = deg
    yn_ref[...] = jnp.dot(cur, w_ref[...], preferred_element_type=jnp.float32)

  return pl.pallas_call(
      k,
      out_shape=(
          jax.ShapeDtypeStruct((_N, 32), jnp.float32),
          jax.ShapeDtypeStruct((_N, w_next.shape[1]), jnp.float32),
          jax.ShapeDtypeStruct((_N, 1), jnp.float32),
      ))(mp, y, b, w_next)


def _tc_layer(mp, y, b, deg, w_next):
  def k(mp_ref, y_ref, b_ref, deg_ref, w_ref, cur_ref, yn_ref):
    t = mp_ref[0, :_N] + mp_ref[1, :_N] + y_ref[...] + b_ref[...]
    cur = jnp.tanh(t / deg_ref[...])
    cur_ref[...] = cur
    yn_ref[...] = jnp.dot(cur, w_ref[...], preferred_element_type=jnp.float32)

  return pl.pallas_call(
      k,
      out_shape=(
          jax.ShapeDtypeStruct((_N, 32), jnp.float32),
          jax.ShapeDtypeStruct((_N, w_next.shape[1]), jnp.float32),
      ))(mp, y, b, deg, w_next)


def _tc_readout(cur1, cur2, cur3, mp3, y3, b3, deg, seg,
                a1, a2, a3, a4, bp, wr, br):
  def k(c1, c2, c3, mp_ref, y3_ref, b3_ref, deg_ref, seg_ref,
        a1r, a2r, a3r, a4r, bpr, wrr, brr, o_ref):
    t4 = (mp_ref[0, :_N, 0:1] + mp_ref[1, :_N, 0:1]
          + y3_ref[:, 0:1] + b3_ref[...])
    cur4 = jnp.tanh(t4 / deg_ref[...])
    t = jnp.dot(c1[...], a1r[...], preferred_element_type=jnp.float32)
    t = t + jnp.dot(c2[...], a2r[...], preferred_element_type=jnp.float32)
    t = t + jnp.dot(c3[...], a3r[...], preferred_element_type=jnp.float32)
    t = t + cur4 * a4r[...] + bpr[...]
    t = jnp.maximum(t, 0.0)
    pooled = jnp.dot(seg_ref[...], t, preferred_element_type=jnp.float32)
    o_ref[...] = jnp.dot(pooled, wrr[...],
                         preferred_element_type=jnp.float32) + brr[...]

  return pl.pallas_call(
      k,
      out_shape=jax.ShapeDtypeStruct((_B, _OUT), jnp.float32),
  )(cur1, cur2, cur3, mp3, y3, b3, deg, seg, a1, a2, a3, a4, bp, wr, br)


@jax.jit
def kernel(node_feat, edge_index, W0, b0, W1, b1, W2, b2, W3, b3,
           W_phi, b_phi, W_rho, b_rho):
  src3 = edge_index[0].reshape(_NW, _CH, _C)
  dst3 = edge_index[1].reshape(_NW, _CH, _C)
  idx4 = jnp.stack([src3, dst3], axis=2)
  z40 = jnp.zeros((_NP, 40), jnp.float32)
  z32 = jnp.zeros((_NP, 32), jnp.float32)
  z8 = jnp.zeros((_NP, 8), jnp.float32)

  y0a = _tc_first(node_feat, W0)
  mp0 = _sc_spmm(y0a, idx4, z40, 40)
  cur1, y1, deg = _tc_layer1(mp0, y0a, b0.reshape(1, 32), W1)

  mp1 = _sc_spmm(y1, idx4, z32, 32)
  cur2, y2 = _tc_layer(mp1, y1, b1.reshape(1, 32), deg, W2)
  mp2 = _sc_spmm(y2, idx4, z32, 32)
  W3p = jnp.pad(W3, ((0, 0), (0, 7)))
  cur3, y3 = _tc_layer(mp2, y2, b2.reshape(1, 32), deg, W3p)

  mp3 = _sc_spmm(y3, idx4, z8, 8)
  seg = jnp.repeat(jnp.eye(_B, dtype=jnp.float32), _G, axis=1)
  return _tc_readout(
      cur1, cur2, cur3, mp3, y3, b3.reshape(1, 1), deg, seg,
      W_phi[0:32], W_phi[32:64], W_phi[64:96], W_phi[96:97].reshape(1, _K),
      b_phi.reshape(1, _K), W_rho, b_rho.reshape(1, _OUT))


# speedup vs baseline: 19.2531x; 1.2787x over previous
import functools

import jax
import jax.numpy as jnp
from jax import lax
from jax.experimental import pallas as pl
from jax.experimental.pallas import tpu as pltpu
from jax.experimental.pallas import tpu_sc as plsc

_N = 10000
_E = 320000
_B = 20
_G = _N // _B
_K = 30
_OUT = 64

_NC = 2
_NS = 16
_NW = _NC * _NS
_C = 125
_CH = _E // (_NW * _C)
_PK = 4
_SJ = _CH // _PK
_NP = 10240
_RPT = _NP // _NS


def _sc_spmm(y_tab, idx4, zrow, width):
  mesh = plsc.VectorSubcoreMesh(core_axis_name="c", subcore_axis_name="s")

  @functools.partial(
      pl.kernel,
      out_type=jax.ShapeDtypeStruct((_NC, _NP, width), jnp.float32),
      mesh=mesh,
      scratch_types=[
          pltpu.VMEM_SHARED((_NP, width), jnp.float32),
          pltpu.VMEM((_CH, 2, _C), jnp.int32),
          pltpu.VMEM((2, _PK, _C, width), jnp.float32),
          pltpu.SemaphoreType.DMA,
          pltpu.SemaphoreType.DMA,
      ],
      compiler_params=pltpu.CompilerParams(use_tc_tiling_on_sc=False),
  )
  def k(y_hbm, idx_hbm, z_hbm, out_hbm, msg_sh, idx_v, rows_v, gsem, ssem):
    c = lax.axis_index("c")
    s = lax.axis_index("s")
    wid = c * _NS + s
    pltpu.sync_copy(idx_hbm.at[wid], idx_v)
    pltpu.sync_copy(z_hbm.at[pl.ds(s * _RPT, _RPT)],
                    msg_sh.at[pl.ds(s * _RPT, _RPT)])
    plsc.subcore_barrier()

    def body(sj, carry):
      p = lax.rem(sj, 2)

      @pl.when(sj >= 2)
      def _():
        for b in range(_PK):
          pltpu.make_async_copy(
              y_hbm.at[pl.ds(0, _C)], rows_v.at[p, b], ssem).wait()

      gds = []
      for b in range(_PK):
        gds.append(pltpu.async_copy(
            y_hbm.at[idx_v.at[sj * _PK + b, 0]], rows_v.at[p, b], gsem))
      for d in gds:
        d.wait()
      for b in range(_PK):
        pltpu.async_copy(
            rows_v.at[p, b], msg_sh.at[idx_v.at[sj * _PK + b, 1]], ssem,
            add=True)
      return carry

    lax.fori_loop(0, _SJ, body, 0)
    for b in range(2 * _PK):
      pltpu.make_async_copy(
          y_hbm.at[pl.ds(0, _C)], rows_v.at[0, b % _PK], ssem).wait()
    plsc.subcore_barrier()
    pltpu.sync_copy(msg_sh.at[pl.ds(s * _RPT, _RPT)],
                    out_hbm.at[c, pl.ds(s * _RPT, _RPT)])

  return k(y_tab, idx4, zrow)


def _tc_first(x, w0):
  def k(x_ref, w_ref, o_ref):
    y = jnp.dot(x_ref[...], w_ref[...], preferred_element_type=jnp.float32)
    o_ref[...] = jnp.concatenate(
        [y,
         jnp.ones((_N, 1), jnp.float32),
         jnp.zeros((_N, 7), jnp.float32)], axis=1)

  return pl.pallas_call(
      k, out_shape=jax.ShapeDtypeStruct((_N, 40), jnp.float32))(x, w0)


def _tc_layer1(mp, y, b, w_next):
  def k(mp_ref, y_ref, b_ref, w_ref, cur_ref, yn_ref, deg_ref):
    deg = mp_ref[0, :_N, 32:33] + mp_ref[1, :_N, 32:33] + 1.0
    t = (mp_ref[0, :_N, :32] + mp_ref[1, :_N, :32]
         + y_ref[:, :32] + b_ref[...])
    cur = jnp.tanh(t / deg)
    cur_ref[...] = cur
    deg_ref[...] = deg
    yn_ref[...] = jnp.dot(cur, w_ref[...], preferred_element_type=jnp.float32)

  return pl.pallas_call(
      k,
      out_shape=(
          jax.ShapeDtypeStruct((_N, 32), jnp.float32),
          jax.ShapeDtypeStruct((_N, w_next.shape[1]), jnp.float32),
          jax.ShapeDtypeStruct((_N, 1), jnp.float32),
      ))(mp, y, b, w_next)


def _tc_layer(mp, y, b, deg, w_next):
  def k(mp_ref, y_ref, b_ref, deg_ref, w_ref, cur_ref, yn_ref):
    t = mp_ref[0, :_N] + mp_ref[1, :_N] + y_ref[...] + b_ref[...]
    cur = jnp.tanh(t / deg_ref[...])
    cur_ref[...] = cur
    yn_ref[...] = jnp.dot(cur, w_ref[...], preferred_element_type=jnp.float32)

  return pl.pallas_call(
      k,
      out_shape=(
          jax.ShapeDtypeStruct((_N, 32), jnp.float32),
          jax.ShapeDtypeStruct((_N, w_next.shape[1]), jnp.float32),
      ))(mp, y, b, deg, w_next)


def _tc_readout(cur1, cur2, cur3, mp3, y3, b3, deg, seg,
                a1, a2, a3, a4, bp, wr, br):
  def k(c1, c2, c3, mp_ref, y3_ref, b3_ref, deg_ref, seg_ref,
        a1r, a2r, a3r, a4r, bpr, wrr, brr, o_ref):
    t4 = (mp_ref[0, :_N, 0:1] + mp_ref[1, :_N, 0:1]
          + y3_ref[:, 0:1] + b3_ref[...])
    cur4 = jnp.tanh(t4 / deg_ref[...])
    t = jnp.dot(c1[...], a1r[...], preferred_element_type=jnp.float32)
    t = t + jnp.dot(c2[...], a2r[...], preferred_element_type=jnp.float32)
    t = t + jnp.dot(c3[...], a3r[...], preferred_element_type=jnp.float32)
    t = t + cur4 * a4r[...] + bpr[...]
    t = jnp.maximum(t, 0.0)
    pooled = jnp.dot(seg_ref[...], t, preferred_element_type=jnp.float32)
    o_ref[...] = jnp.dot(pooled, wrr[...],
                         preferred_element_type=jnp.float32) + brr[...]

  return pl.pallas_call(
      k,
      out_shape=jax.ShapeDtypeStruct((_B, _OUT), jnp.float32),
  )(cur1, cur2, cur3, mp3, y3, b3, deg, seg, a1, a2, a3, a4, bp, wr, br)


@jax.jit
def kernel(node_feat, edge_index, W0, b0, W1, b1, W2, b2, W3, b3,
           W_phi, b_phi, W_rho, b_rho):
  src3 = edge_index[0].reshape(_NW, _CH, _C)
  dst3 = edge_index[1].reshape(_NW, _CH, _C)
  idx4 = jnp.stack([src3, dst3], axis=2)
  z40 = jnp.zeros((_NP, 40), jnp.float32)
  z32 = jnp.zeros((_NP, 32), jnp.float32)
  z8 = jnp.zeros((_NP, 8), jnp.float32)

  y0a = _tc_first(node_feat, W0)
  mp0 = _sc_spmm(y0a, idx4, z40, 40)
  cur1, y1, deg = _tc_layer1(mp0, y0a, b0.reshape(1, 32), W1)

  mp1 = _sc_spmm(y1, idx4, z32, 32)
  cur2, y2 = _tc_layer(mp1, y1, b1.reshape(1, 32), deg, W2)
  mp2 = _sc_spmm(y2, idx4, z32, 32)
  W3p = jnp.pad(W3, ((0, 0), (0, 7)))
  cur3, y3 = _tc_layer(mp2, y2, b2.reshape(1, 32), deg, W3p)

  mp3 = _sc_spmm(y3, idx4, z8, 8)
  seg = jnp.repeat(jnp.eye(_B, dtype=jnp.float32), _G, axis=1)
  return _tc_readout(
      cur1, cur2, cur3, mp3, y3, b3.reshape(1, 1), deg, seg,
      W_phi[0:32], W_phi[32:64], W_phi[64:96], W_phi[96:97].reshape(1, _K),
      b_phi.reshape(1, _K), W_rho, b_rho.reshape(1, _OUT))


# speedup vs baseline: 20.3772x; 1.0584x over previous
import functools

import jax
import jax.numpy as jnp
from jax import lax
from jax.experimental import pallas as pl
from jax.experimental.pallas import tpu as pltpu
from jax.experimental.pallas import tpu_sc as plsc

_N = 10000
_E = 320000
_B = 20
_G = _N // _B
_K = 30
_OUT = 64

_NC = 2
_NS = 16
_NW = _NC * _NS
_C = 500
_CH = _E // (_NW * _C)
_PK = 2
_SJ = _CH // _PK
_NP = 10240
_RPT = _NP // _NS


def _sc_spmm(y_tab, idx4, zrow, width):
  mesh = plsc.VectorSubcoreMesh(core_axis_name="c", subcore_axis_name="s")

  @functools.partial(
      pl.kernel,
      out_type=jax.ShapeDtypeStruct((_NC, _NP, width), jnp.float32),
      mesh=mesh,
      scratch_types=[
          pltpu.VMEM_SHARED((_NP, width), jnp.float32),
          pltpu.VMEM((_CH, 2, _C), jnp.int32),
          pltpu.VMEM((2, _PK, _C, width), jnp.float32),
          pltpu.SemaphoreType.DMA,
          pltpu.SemaphoreType.DMA,
      ],
      compiler_params=pltpu.CompilerParams(use_tc_tiling_on_sc=False),
  )
  def k(y_hbm, idx_hbm, z_hbm, out_hbm, msg_sh, idx_v, rows_v, gsem, ssem):
    c = lax.axis_index("c")
    s = lax.axis_index("s")
    wid = c * _NS + s
    pltpu.sync_copy(idx_hbm.at[wid], idx_v)
    pltpu.sync_copy(z_hbm.at[pl.ds(s * _RPT, _RPT)],
                    msg_sh.at[pl.ds(s * _RPT, _RPT)])
    plsc.subcore_barrier()

    def body(sj, carry):
      p = lax.rem(sj, 2)

      @pl.when(sj >= 2)
      def _():
        for b in range(_PK):
          pltpu.make_async_copy(
              y_hbm.at[pl.ds(0, _C)], rows_v.at[p, b], ssem).wait()

      gds = []
      for b in range(_PK):
        gds.append(pltpu.async_copy(
            y_hbm.at[idx_v.at[sj * _PK + b, 0]], rows_v.at[p, b], gsem))
      for d in gds:
        d.wait()
      for b in range(_PK):
        pltpu.async_copy(
            rows_v.at[p, b], msg_sh.at[idx_v.at[sj * _PK + b, 1]], ssem,
            add=True)
      return carry

    lax.fori_loop(0, _SJ, body, 0)
    for b in range(2 * _PK):
      pltpu.make_async_copy(
          y_hbm.at[pl.ds(0, _C)], rows_v.at[0, b % _PK], ssem).wait()
    plsc.subcore_barrier()
    pltpu.sync_copy(msg_sh.at[pl.ds(s * _RPT, _RPT)],
                    out_hbm.at[c, pl.ds(s * _RPT, _RPT)])

  return k(y_tab, idx4, zrow)


def _tc_first(x, w0):
  def k(x_ref, w_ref, o_ref):
    y = jnp.dot(x_ref[...], w_ref[...], preferred_element_type=jnp.float32)
    o_ref[...] = jnp.concatenate(
        [y,
         jnp.ones((_N, 1), jnp.float32),
         jnp.zeros((_N, 7), jnp.float32)], axis=1)

  return pl.pallas_call(
      k, out_shape=jax.ShapeDtypeStruct((_N, 40), jnp.float32))(x, w0)


def _tc_layer1(mp, y, b, w_next):
  def k(mp_ref, y_ref, b_ref, w_ref, cur_ref, yn_ref, deg_ref):
    deg = mp_ref[0, :_N, 32:33] + mp_ref[1, :_N, 32:33] + 1.0
    t = (mp_ref[0, :_N, :32] + mp_ref[1, :_N, :32]
         + y_ref[:, :32] + b_ref[...])
    cur = jnp.tanh(t / deg)
    cur_ref[...] = cur
    deg_ref[...] = deg
    yn_ref[...] = jnp.dot(cur, w_ref[...], preferred_element_type=jnp.float32)

  return pl.pallas_call(
      k,
      out_shape=(
          jax.ShapeDtypeStruct((_N, 32), jnp.float32),
          jax.ShapeDtypeStruct((_N, w_next.shape[1]), jnp.float32),
          jax.ShapeDtypeStruct((_N, 1), jnp.float32),
      ))(mp, y, b, w_next)


def _tc_layer(mp, y, b, deg, w_next):
  def k(mp_ref, y_ref, b_ref, deg_ref, w_ref, cur_ref, yn_ref):
    t = mp_ref[0, :_N] + mp_ref[1, :_N] + y_ref[...] + b_ref[...]
    cur = jnp.tanh(t / deg_ref[...])
    cur_ref[...] = cur
    yn_ref[...] = jnp.dot(cur, w_ref[...], preferred_element_type=jnp.float32)

  return pl.pallas_call(
      k,
      out_shape=(
          jax.ShapeDtypeStruct((_N, 32), jnp.float32),
          jax.ShapeDtypeStruct((_N, w_next.shape[1]), jnp.float32),
      ))(mp, y, b, deg, w_next)


def _tc_readout(cur1, cur2, cur3, mp3, y3, b3, deg, seg,
                a1, a2, a3, a4, bp, wr, br):
  def k(c1, c2, c3, mp_ref, y3_ref, b3_ref, deg_ref, seg_ref,
        a1r, a2r, a3r, a4r, bpr, wrr, brr, o_ref):
    t4 = (mp_ref[0, :_N, 0:1] + mp_ref[1, :_N, 0:1]
          + y3_ref[:, 0:1] + b3_ref[...])
    cur4 = jnp.tanh(t4 / deg_ref[...])
    t = jnp.dot(c1[...], a1r[...], preferred_element_type=jnp.float32)
    t = t + jnp.dot(c2[...], a2r[...], preferred_element_type=jnp.float32)
    t = t + jnp.dot(c3[...], a3r[...], preferred_element_type=jnp.float32)
    t = t + cur4 * a4r[...] + bpr[...]
    t = jnp.maximum(t, 0.0)
    pooled = jnp.dot(seg_ref[...], t, preferred_element_type=jnp.float32)
    o_ref[...] = jnp.dot(pooled, wrr[...],
                         preferred_element_type=jnp.float32) + brr[...]

  return pl.pallas_call(
      k,
      out_shape=jax.ShapeDtypeStruct((_B, _OUT), jnp.float32),
  )(cur1, cur2, cur3, mp3, y3, b3, deg, seg, a1, a2, a3, a4, bp, wr, br)


@jax.jit
def kernel(node_feat, edge_index, W0, b0, W1, b1, W2, b2, W3, b3,
           W_phi, b_phi, W_rho, b_rho):
  src3 = edge_index[0].reshape(_NW, _CH, _C)
  dst3 = edge_index[1].reshape(_NW, _CH, _C)
  idx4 = jnp.stack([src3, dst3], axis=2)
  z40 = jnp.zeros((_NP, 40), jnp.float32)
  z32 = jnp.zeros((_NP, 32), jnp.float32)
  z8 = jnp.zeros((_NP, 8), jnp.float32)

  y0a = _tc_first(node_feat, W0)
  mp0 = _sc_spmm(y0a, idx4, z40, 40)
  cur1, y1, deg = _tc_layer1(mp0, y0a, b0.reshape(1, 32), W1)

  mp1 = _sc_spmm(y1, idx4, z32, 32)
  cur2, y2 = _tc_layer(mp1, y1, b1.reshape(1, 32), deg, W2)
  mp2 = _sc_spmm(y2, idx4, z32, 32)
  W3p = jnp.pad(W3, ((0, 0), (0, 7)))
  cur3, y3 = _tc_layer(mp2, y2, b2.reshape(1, 32), deg, W3p)

  mp3 = _sc_spmm(y3, idx4, z8, 8)
  seg = jnp.repeat(jnp.eye(_B, dtype=jnp.float32), _G, axis=1)
  return _tc_readout(
      cur1, cur2, cur3, mp3, y3, b3.reshape(1, 1), deg, seg,
      W_phi[0:32], W_phi[32:64], W_phi[64:96], W_phi[96:97].reshape(1, _K),
      b_phi.reshape(1, _K), W_rho, b_rho.reshape(1, _OUT))


# speedup vs baseline: 21.3940x; 1.0499x over previous
import functools

import jax
import jax.numpy as jnp
from jax import lax
from jax.experimental import pallas as pl
from jax.experimental.pallas import tpu as pltpu
from jax.experimental.pallas import tpu_sc as plsc

_N = 10000
_E = 320000
_B = 20
_G = _N // _B
_K = 30
_OUT = 64

_NC = 2
_NS = 16
_NW = _NC * _NS
_C = 1000
_CH = _E // (_NW * _C)
_PK = 1
_SJ = _CH // _PK
_NP = 10240
_RPT = _NP // _NS


def _sc_spmm(y_tab, src3, dst3, zrow, width):
  mesh = plsc.VectorSubcoreMesh(core_axis_name="c", subcore_axis_name="s")

  @functools.partial(
      pl.kernel,
      out_type=jax.ShapeDtypeStruct((_NC, _NP, width), jnp.float32),
      mesh=mesh,
      scratch_types=[
          pltpu.VMEM_SHARED((_NP, width), jnp.float32),
          pltpu.VMEM((_CH, _C), jnp.int32),
          pltpu.VMEM((_CH, _C), jnp.int32),
          pltpu.VMEM((2, _PK, _C, width), jnp.float32),
          pltpu.SemaphoreType.DMA,
          pltpu.SemaphoreType.DMA,
      ],
      compiler_params=pltpu.CompilerParams(use_tc_tiling_on_sc=False),
  )
  def k(y_hbm, src_hbm, dst_hbm, z_hbm, out_hbm, msg_sh, src_v, dst_v,
        rows_v, gsem, ssem):
    c = lax.axis_index("c")
    s = lax.axis_index("s")
    wid = c * _NS + s
    pltpu.sync_copy(src_hbm.at[wid], src_v)
    pltpu.sync_copy(dst_hbm.at[wid], dst_v)
    pltpu.sync_copy(z_hbm.at[pl.ds(s * _RPT, _RPT)],
                    msg_sh.at[pl.ds(s * _RPT, _RPT)])
    plsc.subcore_barrier()

    def body(sj, carry):
      p = lax.rem(sj, 2)

      @pl.when(sj >= 2)
      def _():
        for b in range(_PK):
          pltpu.make_async_copy(
              y_hbm.at[pl.ds(0, _C)], rows_v.at[p, b], ssem).wait()

      gds = []
      for b in range(_PK):
        gds.append(pltpu.async_copy(
            y_hbm.at[src_v.at[sj * _PK + b]], rows_v.at[p, b], gsem))
      for d in gds:
        d.wait()
      for b in range(_PK):
        pltpu.async_copy(
            rows_v.at[p, b], msg_sh.at[dst_v.at[sj * _PK + b]], ssem,
            add=True)
      return carry

    lax.fori_loop(0, _SJ, body, 0)
    for b in range(2 * _PK):
      pltpu.make_async_copy(
          y_hbm.at[pl.ds(0, _C)], rows_v.at[0, b % _PK], ssem).wait()
    plsc.subcore_barrier()
    pltpu.sync_copy(msg_sh.at[pl.ds(s * _RPT, _RPT)],
                    out_hbm.at[c, pl.ds(s * _RPT, _RPT)])

  return k(y_tab, src3, dst3, zrow)


def _tc_first(x, w0):
  def k(x_ref, w_ref, o_ref):
    y = jnp.dot(x_ref[...], w_ref[...], preferred_element_type=jnp.float32)
    o_ref[...] = jnp.concatenate(
        [y,
         jnp.ones((_N, 1), jnp.float32),
         jnp.zeros((_N, 7), jnp.float32)], axis=1)

  return pl.pallas_call(
      k, out_shape=jax.ShapeDtypeStruct((_N, 40), jnp.float32))(x, w0)


def _tc_layer1(mp, y, b, w_next):
  def k(mp_ref, y_ref, b_ref, w_ref, cur_ref, yn_ref, deg_ref):
    deg = mp_ref[0, :_N, 32:33] + mp_ref[1, :_N, 32:33] + 1.0
    t = (mp_ref[0, :_N, :32] + mp_ref[1, :_N, :32]
         + y_ref[:, :32] + b_ref[...])
    cur = jnp.tanh(t / deg)
    cur_ref[...] = cur
    deg_ref[...] = deg
    yn_ref[...] = jnp.dot(cur, w_ref[...], preferred_element_type=jnp.float32)

  return pl.pallas_call(
      k,
      out_shape=(
          jax.ShapeDtypeStruct((_N, 32), jnp.float32),
          jax.ShapeDtypeStruct((_N, w_next.shape[1]), jnp.float32),
          jax.ShapeDtypeStruct((_N, 1), jnp.float32),
      ))(mp, y, b, w_next)


def _tc_layer(mp, y, b, deg, w_next):
  def k(mp_ref, y_ref, b_ref, deg_ref, w_ref, cur_ref, yn_ref):
    t = mp_ref[0, :_N] + mp_ref[1, :_N] + y_ref[...] + b_ref[...]
    cur = jnp.tanh(t / deg_ref[...])
    cur_ref[...] = cur
    yn_ref[...] = jnp.dot(cur, w_ref[...], preferred_element_type=jnp.float32)

  return pl.pallas_call(
      k,
      out_shape=(
          jax.ShapeDtypeStruct((_N, 32), jnp.float32),
          jax.ShapeDtypeStruct((_N, w_next.shape[1]), jnp.float32),
      ))(mp, y, b, deg, w_next)


def _tc_readout(cur1, cur2, cur3, mp3, y3, b3, deg, seg, wphi, bp, wr, br):
  def k(c1, c2, c3, mp_ref, y3_ref, b3_ref, deg_ref, seg_ref,
        wphi_ref, bpr, wrr, brr, o_ref):
    t4 = (mp_ref[0, :_N, 0:1] + mp_ref[1, :_N, 0:1]
          + y3_ref[:, 0:1] + b3_ref[...])
    cur4 = jnp.tanh(t4 / deg_ref[...])
    t = jnp.dot(c1[...], wphi_ref[0:32],
                preferred_element_type=jnp.float32)
    t = t + jnp.dot(c2[...], wphi_ref[32:64],
                    preferred_element_type=jnp.float32)
    t = t + jnp.dot(c3[...], wphi_ref[64:96],
                    preferred_element_type=jnp.float32)
    t = t + cur4 * wphi_ref[96:97] + bpr[...]
    t = jnp.maximum(t, 0.0)
    pooled = jnp.dot(seg_ref[...], t, preferred_element_type=jnp.float32)
    o_ref[...] = jnp.dot(pooled, wrr[...],
                         preferred_element_type=jnp.float32) + brr[...]

  return pl.pallas_call(
      k,
      out_shape=jax.ShapeDtypeStruct((_B, _OUT), jnp.float32),
  )(cur1, cur2, cur3, mp3, y3, b3, deg, seg, wphi, bp, wr, br)


@jax.jit
def kernel(node_feat, edge_index, W0, b0, W1, b1, W2, b2, W3, b3,
           W_phi, b_phi, W_rho, b_rho):
  src3 = edge_index[0].reshape(_NW, _CH, _C)
  dst3 = edge_index[1].reshape(_NW, _CH, _C)
  z40 = jnp.zeros((_NP, 40), jnp.float32)
  z32 = jnp.zeros((_NP, 32), jnp.float32)
  z8 = jnp.zeros((_NP, 8), jnp.float32)

  y0a = _tc_first(node_feat, W0)
  mp0 = _sc_spmm(y0a, src3, dst3, z40, 40)
  cur1, y1, deg = _tc_layer1(mp0, y0a, b0.reshape(1, 32), W1)

  mp1 = _sc_spmm(y1, src3, dst3, z32, 32)
  cur2, y2 = _tc_layer(mp1, y1, b1.reshape(1, 32), deg, W2)
  mp2 = _sc_spmm(y2, src3, dst3, z32, 32)
  W3p = jnp.pad(W3, ((0, 0), (0, 7)))
  cur3, y3 = _tc_layer(mp2, y2, b2.reshape(1, 32), deg, W3p)

  mp3 = _sc_spmm(y3, src3, dst3, z8, 8)
  seg = jnp.repeat(jnp.eye(_B, dtype=jnp.float32), _G, axis=1)
  return _tc_readout(
      cur1, cur2, cur3, mp3, y3, b3.reshape(1, 1), deg, seg,
      W_phi, b_phi.reshape(1, _K), W_rho, b_rho.reshape(1, _OUT))


# speedup vs baseline: 21.4402x; 1.0022x over previous
import functools

import jax
import jax.numpy as jnp
from jax import lax
from jax.experimental import pallas as pl
from jax.experimental.pallas import tpu as pltpu
from jax.experimental.pallas import tpu_sc as plsc

_N = 10000
_E = 320000
_B = 20
_G = _N // _B
_K = 30
_OUT = 64

_NC = 2
_NS = 16
_NW = _NC * _NS
_C = 1000
_CH = _E // (_NW * _C)
_PK = 1
_SJ = _CH // _PK
_NP = 10240
_RPT = _NP // _NS


def _sc_spmm(y_tab, src3, dst3, zrow, width):
  mesh = plsc.VectorSubcoreMesh(core_axis_name="c", subcore_axis_name="s")

  @functools.partial(
      pl.kernel,
      out_type=jax.ShapeDtypeStruct((_NC, _NP, width), jnp.float32),
      mesh=mesh,
      scratch_types=[
          pltpu.VMEM_SHARED((_NP, width), jnp.float32),
          pltpu.VMEM((_CH, _C), jnp.int32),
          pltpu.VMEM((_CH, _C), jnp.int32),
          pltpu.VMEM((2, _PK, _C, width), jnp.float32),
          pltpu.SemaphoreType.DMA,
          pltpu.SemaphoreType.DMA,
      ],
      compiler_params=pltpu.CompilerParams(use_tc_tiling_on_sc=False),
  )
  def k(y_hbm, src_hbm, dst_hbm, z_hbm, out_hbm, msg_sh, src_v, dst_v,
        rows_v, gsem, ssem):
    c = lax.axis_index("c")
    s = lax.axis_index("s")
    wid = c * _NS + s
    pltpu.sync_copy(src_hbm.at[wid], src_v)
    pltpu.sync_copy(dst_hbm.at[wid], dst_v)
    pltpu.sync_copy(z_hbm.at[pl.ds(s * _RPT, _RPT)],
                    msg_sh.at[pl.ds(s * _RPT, _RPT)])
    plsc.subcore_barrier()

    def body(sj, carry):
      p = lax.rem(sj, 2)

      @pl.when(sj >= 2)
      def _():
        for b in range(_PK):
          pltpu.make_async_copy(
              y_hbm.at[pl.ds(0, _C)], rows_v.at[p, b], ssem).wait()

      gds = []
      for b in range(_PK):
        gds.append(pltpu.async_copy(
            y_hbm.at[src_v.at[sj * _PK + b]], rows_v.at[p, b], gsem))
      for d in gds:
        d.wait()
      for b in range(_PK):
        pltpu.async_copy(
            rows_v.at[p, b], msg_sh.at[dst_v.at[sj * _PK + b]], ssem,
            add=True)
      return carry

    lax.fori_loop(0, _SJ, body, 0)
    for b in range(2 * _PK):
      pltpu.make_async_copy(
          y_hbm.at[pl.ds(0, _C)], rows_v.at[0, b % _PK], ssem).wait()
    plsc.subcore_barrier()
    pltpu.sync_copy(msg_sh.at[pl.ds(s * _RPT, _RPT)],
                    out_hbm.at[c, pl.ds(s * _RPT, _RPT)])

  return k(y_tab, src3, dst3, zrow)


def _tc_first(x, w0):
  def k(x_ref, w_ref, o_ref):
    y = jnp.dot(x_ref[...], w_ref[...], preferred_element_type=jnp.float32)
    o_ref[...] = jnp.concatenate(
        [y,
         jnp.ones((_N, 1), jnp.float32),
         jnp.zeros((_N, 7), jnp.float32)], axis=1)

  return pl.pallas_call(
      k, out_shape=jax.ShapeDtypeStruct((_N, 40), jnp.float32))(x, w0)


def _tc_layer1(mp, y, b, w_next):
  def k(mp_ref, y_ref, b_ref, w_ref, cur_ref, yn_ref, inv_ref):
    inv = 1.0 / (mp_ref[0, :_N, 32:33] + mp_ref[1, :_N, 32:33] + 1.0)
    t = (mp_ref[0, :_N, :32] + mp_ref[1, :_N, :32]
         + y_ref[:, :32] + b_ref[...])
    cur = jnp.tanh(t * inv)
    cur_ref[...] = cur
    inv_ref[...] = inv
    yn_ref[...] = jnp.dot(cur, w_ref[...], preferred_element_type=jnp.float32)

  return pl.pallas_call(
      k,
      out_shape=(
          jax.ShapeDtypeStruct((_N, 32), jnp.float32),
          jax.ShapeDtypeStruct((_N, w_next.shape[1]), jnp.float32),
          jax.ShapeDtypeStruct((_N, 1), jnp.float32),
      ))(mp, y, b, w_next)


def _tc_layer(mp, y, b, inv, w_next):
  def k(mp_ref, y_ref, b_ref, inv_ref, w_ref, cur_ref, yn_ref):
    t = mp_ref[0, :_N] + mp_ref[1, :_N] + y_ref[...] + b_ref[...]
    cur = jnp.tanh(t * inv_ref[...])
    cur_ref[...] = cur
    yn_ref[...] = jnp.dot(cur, w_ref[...], preferred_element_type=jnp.float32)

  return pl.pallas_call(
      k,
      out_shape=(
          jax.ShapeDtypeStruct((_N, 32), jnp.float32),
          jax.ShapeDtypeStruct((_N, w_next.shape[1]), jnp.float32),
      ))(mp, y, b, inv, w_next)


def _tc_phi123(cur1, cur2, cur3, wphi, bp):
  def k(c1, c2, c3, wphi_ref, bpr, o_ref):
    t = jnp.dot(c1[...], wphi_ref[0:32],
                preferred_element_type=jnp.float32)
    t = t + jnp.dot(c2[...], wphi_ref[32:64],
                    preferred_element_type=jnp.float32)
    t = t + jnp.dot(c3[...], wphi_ref[64:96],
                    preferred_element_type=jnp.float32)
    o_ref[...] = t + bpr[...]

  return pl.pallas_call(
      k,
      out_shape=jax.ShapeDtypeStruct((_N, _K), jnp.float32),
  )(cur1, cur2, cur3, wphi, bp)


def _tc_readout(s123, mp3, y3, b3, inv, seg, wphi, wr, br):
  def k(s_ref, mp_ref, y3_ref, b3_ref, inv_ref, seg_ref,
        wphi_ref, wrr, brr, o_ref):
    t4 = (mp_ref[0, :_N, 0:1] + mp_ref[1, :_N, 0:1]
          + y3_ref[:, 0:1] + b3_ref[...])
    cur4 = jnp.tanh(t4 * inv_ref[...])
    t = jnp.maximum(s_ref[...] + cur4 * wphi_ref[96:97], 0.0)
    pooled = jnp.dot(seg_ref[...], t, preferred_element_type=jnp.float32)
    o_ref[...] = jnp.dot(pooled, wrr[...],
                         preferred_element_type=jnp.float32) + brr[...]

  return pl.pallas_call(
      k,
      out_shape=jax.ShapeDtypeStruct((_B, _OUT), jnp.float32),
  )(s123, mp3, y3, b3, inv, seg, wphi, wr, br)


@jax.jit
def kernel(node_feat, edge_index, W0, b0, W1, b1, W2, b2, W3, b3,
           W_phi, b_phi, W_rho, b_rho):
  src3 = edge_index[0].reshape(_NW, _CH, _C)
  dst3 = edge_index[1].reshape(_NW, _CH, _C)
  z40 = jnp.zeros((_NP, 40), jnp.float32)
  z32 = jnp.zeros((_NP, 32), jnp.float32)
  z8 = jnp.zeros((_NP, 8), jnp.float32)

  y0a = _tc_first(node_feat, W0)
  mp0 = _sc_spmm(y0a, src3, dst3, z40, 40)
  cur1, y1, inv = _tc_layer1(mp0, y0a, b0.reshape(1, 32), W1)

  mp1 = _sc_spmm(y1, src3, dst3, z32, 32)
  cur2, y2 = _tc_layer(mp1, y1, b1.reshape(1, 32), inv, W2)
  mp2 = _sc_spmm(y2, src3, dst3, z32, 32)
  W3p = jnp.pad(W3, ((0, 0), (0, 7)))
  cur3, y3 = _tc_layer(mp2, y2, b2.reshape(1, 32), inv, W3p)

  mp3 = _sc_spmm(y3, src3, dst3, z8, 8)
  s123 = _tc_phi123(cur1, cur2, cur3, W_phi, b_phi.reshape(1, _K))
  seg = jnp.repeat(jnp.eye(_B, dtype=jnp.float32), _G, axis=1)
  return _tc_readout(
      s123, mp3, y3, b3.reshape(1, 1), inv, seg,
      W_phi, W_rho, b_rho.reshape(1, _OUT))
